# baseline (speedup 1.0000x reference)
import functools

import jax
import jax.numpy as jnp
from jax import lax
from jax.experimental import pallas as pl
from jax.experimental.pallas import tpu as pltpu
from jax.experimental.pallas import tpu_sc as plsc

VOCAB = 1000000
DIM = 64
B = 4096
T = 200
VB = (VOCAB + 127) // 128

NUM_CORES = 2
NUM_SUBCORES = 16
NUM_WORKERS = NUM_CORES * NUM_SUBCORES


def _transpose_body(wt_hbm, wrm_hbm, tb, ob, isems, osems):
    w = lax.axis_index("s") * NUM_CORES + lax.axis_index("c")
    lo = (w * VB) // NUM_WORKERS
    hi = ((w + 1) * VB) // NUM_WORKERS
    nb = hi - lo

    iota = lax.iota(jnp.int32, 16)

    def start_in(j, buf):
        for db in range(8):
            pltpu.async_copy(
                wt_hbm.at[pl.ds(db * 8, 8), pl.ds((lo + j) * 128, 128)],
                tb[buf].at[pl.ds(db * 8, 8)], isems[buf])

    def wait_in(buf):
        for db in range(8):
            pltpu.make_async_copy(wt_hbm.at[pl.ds(0, 8), pl.ds(0, 128)],
                                  tb[buf].at[pl.ds(db * 8, 8)],
                                  isems[buf]).wait()

    def start_out(j, buf):
        pltpu.async_copy(ob[buf], wrm_hbm.at[pl.ds((lo + j) * 64, 64)],
                         osems[buf])

    def wait_out(buf):
        pltpu.make_async_copy(ob[buf], wrm_hbm.at[pl.ds(0, 64)],
                              osems[buf]).wait()

    def transpose_group(ibuf, obuf, m):
        blvec = iota + 16 * m
        rowm = lax.shift_right_logical(blvec, 1)
        colm = lax.mul(lax.bitwise_and(blvec, 1), 64)
        rot = iota
        for s in range(16):
            for dg in range(4):
                dvec = rot + dg * 16
                vals = plsc.load_gather(tb[ibuf], [dvec, blvec])
                plsc.store_scatter(ob[obuf], [rowm, colm + dvec], vals)
            rot = lax.bitwise_and(rot + 1, 15)

    for i in range(4):
        start_in(i, i)

    def step(q, carry):
        for par in range(4):
            j = 4 * q + par

            @pl.when(j < nb)
            def _do():
                wait_in(par)

                @pl.when(j >= 2)
                def _wo():
                    wait_out(par % 2)

                def tgroup(m, c2):
                    transpose_group(par, par % 2, m)
                    return c2

                lax.fori_loop(0, 8, tgroup, 0)
                start_out(j, par % 2)

                @pl.when(j + 4 < nb)
                def _pf():
                    start_in(j + 4, par)

        return carry

    lax.fori_loop(0, (nb + 3) // 4, step, 0)

    wait_out(0)
    wait_out(1)


def _lookup_body(textt_hbm, wrm_hbm, out_hbm, txt, pp, fp, rb, ob,
                 gsems, osems):
    w = lax.axis_index("s") * NUM_CORES + lax.axis_index("c")

    iota = lax.iota(jnp.int32, 16)
    rvecs = [iota + 16 * ll for ll in range(8)]

    pltpu.sync_copy(textt_hbm.at[:, pl.ds(w * 128, 128)], txt)

    def compute_idx(t, buf):
        for ll in range(8):
            idxv = txt[t, pl.ds(16 * ll, 16)]
            pp[buf][pl.ds(16 * ll, 16)] = lax.shift_right_logical(idxv, 1)
            fp[buf][pl.ds(16 * ll, 16)] = lax.mul(lax.bitwise_and(idxv, 1),
                                                  64)

    def start_gather(buf):
        pltpu.async_copy(wrm_hbm.at[pp[buf]], rb[buf], gsems[buf])

    def wait_gather(buf):
        pltpu.make_async_copy(wrm_hbm.at[pp[buf]], rb[buf],
                              gsems[buf]).wait()

    def start_out(t, buf):
        pltpu.async_copy(ob[buf], out_hbm.at[t, :, pl.ds(w * 128, 128)],
                         osems[buf])

    def wait_out(buf):
        pltpu.make_async_copy(ob[buf],
                              out_hbm.at[0, :, pl.ds(w * 128, 128)],
                              osems[buf]).wait()

    def transpose_rows(ibuf, obuf):
        cbases = [fp[ibuf][pl.ds(16 * ll, 16)] for ll in range(8)]

        def dgroup(dg, c2):
            dg16 = dg * 16
            rot = iota
            for s in range(16):
                dvec = rot + dg16
                for ll in range(8):
                    vals = plsc.load_gather(rb[ibuf],
                                            [rvecs[ll], cbases[ll] + dvec])
                    plsc.store_scatter(ob[obuf], [dvec, rvecs[ll]], vals)
                rot = lax.bitwise_and(rot + 1, 15)
            return c2

        lax.fori_loop(0, DIM // 16, dgroup, 0)

    for i in range(3):
        compute_idx(i, i)
        start_gather(i)

    def step(q, carry):
        for par in range(4):
            t = 4 * q + par

            @pl.when(t + 3 < T)
            def _pf():
                compute_idx(t + 3, (par + 3) % 4)
                start_gather((par + 3) % 4)

            wait_gather(par)

            @pl.when(t >= 2)
            def _wo():
                wait_out(par % 2)

            transpose_rows(par, par % 2)
            start_out(t, par % 2)

        return carry

    lax.fori_loop(0, T // 4, step, 0)
    wait_out(0)
    wait_out(1)


@jax.jit
def kernel(text, W):
    wt = W.T
    textt = text.T

    k1 = functools.partial(
        pl.kernel,
        mesh=plsc.VectorSubcoreMesh(core_axis_name="c", subcore_axis_name="s"),
        out_type=jax.ShapeDtypeStruct((VB * 64, 128), jnp.float32),
        scratch_types=[
            [pltpu.VMEM((DIM, 128), jnp.float32) for _ in range(4)],
            [pltpu.VMEM((64, 128), jnp.float32) for _ in range(2)],
            [pltpu.SemaphoreType.DMA for _ in range(4)],
            [pltpu.SemaphoreType.DMA for _ in range(2)],
        ],
        compiler_params=pltpu.CompilerParams(
            use_tc_tiling_on_sc=True, needs_layout_passes=False, disable_bounds_checks=True),
    )(_transpose_body)
    w_rm = k1(wt)

    k2 = functools.partial(
        pl.kernel,
        mesh=plsc.VectorSubcoreMesh(core_axis_name="c", subcore_axis_name="s"),
        out_type=jax.ShapeDtypeStruct((T, DIM, B), jnp.float32),
        scratch_types=[
            pltpu.VMEM((T, 128), jnp.int32),
            [pltpu.VMEM((128,), jnp.int32) for _ in range(4)],
            [pltpu.VMEM((128,), jnp.int32) for _ in range(4)],
            [pltpu.VMEM((128, 128), jnp.float32) for _ in range(4)],
            [pltpu.VMEM((DIM, 128), jnp.float32) for _ in range(2)],
            [pltpu.SemaphoreType.DMA for _ in range(4)],
            [pltpu.SemaphoreType.DMA for _ in range(2)],
        ],
        compiler_params=pltpu.CompilerParams(
            use_tc_tiling_on_sc=True, needs_layout_passes=False, disable_bounds_checks=True),
    )(_lookup_body)
    out_t = k2(textt, w_rm)

    return jnp.transpose(out_t, (2, 0, 1))


# speedup vs baseline: 1.0073x; 1.0073x over previous
import functools

import jax
import jax.numpy as jnp
from jax import lax
from jax.experimental import pallas as pl
from jax.experimental.pallas import tpu as pltpu
from jax.experimental.pallas import tpu_sc as plsc

VOCAB = 1000000
DIM = 64
B = 4096
T = 200
VB = (VOCAB + 127) // 128

NUM_CORES = 2
NUM_SUBCORES = 16
NUM_WORKERS = NUM_CORES * NUM_SUBCORES


def _transpose_body(wt_hbm, wrm_hbm, tb, ob, isems, osems):
    w = lax.axis_index("s") * NUM_CORES + lax.axis_index("c")
    lo = (w * VB) // NUM_WORKERS
    hi = ((w + 1) * VB) // NUM_WORKERS
    nb = hi - lo

    iota = lax.iota(jnp.int32, 16)
    rots = [lax.bitwise_and(iota + s, 15) for s in range(16)]

    def start_in(j, buf):
        pltpu.async_copy(wt_hbm.at[:, pl.ds((lo + j) * 128, 128)], tb[buf],
                         isems[buf])

    def wait_in(buf):
        pltpu.make_async_copy(wt_hbm.at[:, pl.ds(0, 128)], tb[buf],
                              isems[buf]).wait()

    def start_out(j, buf):
        pltpu.async_copy(ob[buf], wrm_hbm.at[pl.ds((lo + j) * 64, 64)],
                         osems[buf])

    def wait_out(buf):
        pltpu.make_async_copy(ob[buf], wrm_hbm.at[pl.ds(0, 64)],
                              osems[buf]).wait()

    def transpose_group(ibuf, obuf, m):
        blvec = iota + 16 * m
        rowm = lax.shift_right_logical(blvec, 1)
        colm = lax.mul(lax.bitwise_and(blvec, 1), 64)
        for dg in range(4):
            for s in range(16):
                dvec = rots[s] + dg * 16
                vals = plsc.load_gather(tb[ibuf], [dvec, blvec])
                plsc.store_scatter(ob[obuf], [rowm, colm + dvec], vals)

    for i in range(4):
        start_in(i, i)

    def step(q, carry):
        for par in range(4):
            j = 4 * q + par

            @pl.when(j < nb)
            def _do():
                wait_in(par)

                @pl.when(j >= 2)
                def _wo():
                    wait_out(par % 2)

                def tgroup(m, c2):
                    transpose_group(par, par % 2, m)
                    return c2

                lax.fori_loop(0, 8, tgroup, 0)
                start_out(j, par % 2)

                @pl.when(j + 4 < nb)
                def _pf():
                    start_in(j + 4, par)

        return carry

    lax.fori_loop(0, (nb + 3) // 4, step, 0)

    wait_out(0)
    wait_out(1)


def _lookup_body(textt_hbm, wrm_hbm, out_hbm, txt, pp, fp, rb, ob,
                 gsems, osems):
    w = lax.axis_index("s") * NUM_CORES + lax.axis_index("c")

    iota = lax.iota(jnp.int32, 16)
    rvecs = [iota + 16 * ll for ll in range(8)]
    rots = [lax.bitwise_and(iota + s, 15) for s in range(16)]

    pltpu.sync_copy(textt_hbm.at[:, pl.ds(w * 128, 128)], txt)

    def compute_idx(t, buf):
        for ll in range(8):
            idxv = txt[t, pl.ds(16 * ll, 16)]
            pp[buf][pl.ds(16 * ll, 16)] = lax.shift_right_logical(idxv, 1)
            fp[buf][pl.ds(16 * ll, 16)] = lax.mul(lax.bitwise_and(idxv, 1),
                                                  64)

    def start_gather(buf):
        pltpu.async_copy(wrm_hbm.at[pp[buf]], rb[buf], gsems[buf])

    def wait_gather(buf):
        pltpu.make_async_copy(wrm_hbm.at[pp[buf]], rb[buf],
                              gsems[buf]).wait()

    def start_out(t, buf):
        pltpu.async_copy(ob[buf], out_hbm.at[t, :, pl.ds(w * 128, 128)],
                         osems[buf])

    def wait_out(buf):
        pltpu.make_async_copy(ob[buf],
                              out_hbm.at[0, :, pl.ds(w * 128, 128)],
                              osems[buf]).wait()

    def transpose_rows(ibuf, obuf):
        cbases = [fp[ibuf][pl.ds(16 * ll, 16)] for ll in range(8)]

        def dgroup(dg, c2):
            for s in range(16):
                dvec = rots[s] + dg * 16
                for ll in range(8):
                    vals = plsc.load_gather(rb[ibuf],
                                            [rvecs[ll], cbases[ll] + dvec])
                    plsc.store_scatter(ob[obuf], [dvec, rvecs[ll]], vals)
            return c2

        lax.fori_loop(0, DIM // 16, dgroup, 0)

    for i in range(3):
        compute_idx(i, i)
        start_gather(i)

    def step(q, carry):
        for par in range(4):
            t = 4 * q + par

            @pl.when(t + 3 < T)
            def _pf():
                compute_idx(t + 3, (par + 3) % 4)
                start_gather((par + 3) % 4)

            wait_gather(par)

            @pl.when(t >= 2)
            def _wo():
                wait_out(par % 2)

            transpose_rows(par, par % 2)
            start_out(t, par % 2)

        return carry

    lax.fori_loop(0, T // 4, step, 0)
    wait_out(0)
    wait_out(1)


@jax.jit
def kernel(text, W):
    wt = W.T
    textt = text.T

    k1 = functools.partial(
        pl.kernel,
        mesh=plsc.VectorSubcoreMesh(core_axis_name="c", subcore_axis_name="s"),
        out_type=jax.ShapeDtypeStruct((VB * 64, 128), jnp.float32),
        scratch_types=[
            [pltpu.VMEM((DIM, 128), jnp.float32) for _ in range(4)],
            [pltpu.VMEM((64, 128), jnp.float32) for _ in range(2)],
            [pltpu.SemaphoreType.DMA for _ in range(4)],
            [pltpu.SemaphoreType.DMA for _ in range(2)],
        ],
        compiler_params=pltpu.CompilerParams(
            use_tc_tiling_on_sc=True, needs_layout_passes=False),
    )(_transpose_body)
    w_rm = k1(wt)

    k2 = functools.partial(
        pl.kernel,
        mesh=plsc.VectorSubcoreMesh(core_axis_name="c", subcore_axis_name="s"),
        out_type=jax.ShapeDtypeStruct((T, DIM, B), jnp.float32),
        scratch_types=[
            pltpu.VMEM((T, 128), jnp.int32),
            [pltpu.VMEM((128,), jnp.int32) for _ in range(4)],
            [pltpu.VMEM((128,), jnp.int32) for _ in range(4)],
            [pltpu.VMEM((128, 128), jnp.float32) for _ in range(4)],
            [pltpu.VMEM((DIM, 128), jnp.float32) for _ in range(2)],
            [pltpu.SemaphoreType.DMA for _ in range(4)],
            [pltpu.SemaphoreType.DMA for _ in range(2)],
        ],
        compiler_params=pltpu.CompilerParams(
            use_tc_tiling_on_sc=True, needs_layout_passes=False),
    )(_lookup_body)
    out_t = k2(textt, w_rm)

    return jnp.transpose(out_t, (2, 0, 1))
